# Initial kernel scaffold; baseline (speedup 1.0000x reference)
import jax
import jax.numpy as jnp
from jax.experimental import pallas as pl


def kernel(x, edge_index, W1, b1, W2, b2):
    raise NotImplementedError("write your pallas kernel here")


# speedup vs baseline: 17.2621x; 17.2621x over previous
import functools

import jax
import jax.numpy as jnp
from jax import lax
from jax.experimental import pallas as pl
from jax.experimental.pallas import tpu as pltpu
from jax.experimental.pallas import tpu_sc as plsc

N = 10000
E = 320000
D_HID = 32

NW = 32
CH = 128
NCH = 80
E_PAD = NW * NCH * CH
ROWS_PT = 632
N_ACC = 16 * ROWS_PT

_MESH = plsc.VectorSubcoreMesh(core_axis_name="c", subcore_axis_name="s")


def _prop_body(u_hbm, src_hbm, dst_hbm, zeros_hbm, out_hbm,
               sidx_v, didx_v, rows_v, acc_sh, sem):
    c = lax.axis_index("c")
    s = lax.axis_index("s")
    wid = c * 16 + s
    r0 = s * ROWS_PT
    pltpu.sync_copy(zeros_hbm.at[pl.ds(r0, ROWS_PT)],
                    acc_sh.at[pl.ds(r0, ROWS_PT)])
    plsc.subcore_barrier()

    def body(i, carry):
        pltpu.sync_copy(src_hbm.at[wid, i], sidx_v)
        pltpu.sync_copy(dst_hbm.at[wid, i], didx_v)
        pltpu.async_copy(u_hbm.at[sidx_v], rows_v, sem).wait()
        pltpu.sync_copy(rows_v, acc_sh.at[didx_v], add=True)
        return carry

    lax.fori_loop(0, NCH, body, 0)
    plsc.subcore_barrier()
    pltpu.sync_copy(acc_sh.at[pl.ds(r0, ROWS_PT)],
                    out_hbm.at[c, pl.ds(r0, ROWS_PT)])


_SC_PARAMS = pltpu.CompilerParams(use_tc_tiling_on_sc=False)

_prop = pl.kernel(
    _prop_body,
    mesh=_MESH,
    compiler_params=_SC_PARAMS,
    out_type=jax.ShapeDtypeStruct((2, N_ACC, D_HID), jnp.float32),
    scratch_types=[
        pltpu.VMEM((CH,), jnp.int32),
        pltpu.VMEM((CH,), jnp.int32),
        pltpu.VMEM((CH, D_HID), jnp.float32),
        pltpu.VMEM_SHARED((N_ACC, D_HID), jnp.float32),
        pltpu.SemaphoreType.DMA,
    ],
)


def _deg_body(dst_hbm, ones_hbm, zeros_hbm, out_hbm, didx_v, ones_v, acc_sh):
    c = lax.axis_index("c")
    s = lax.axis_index("s")
    wid = c * 16 + s
    r0 = s * ROWS_PT
    pltpu.sync_copy(zeros_hbm.at[pl.ds(r0, ROWS_PT)],
                    acc_sh.at[pl.ds(r0, ROWS_PT)])
    pltpu.sync_copy(ones_hbm, ones_v)
    plsc.subcore_barrier()

    def body(i, carry):
        pltpu.sync_copy(dst_hbm.at[wid, i], didx_v)
        pltpu.sync_copy(ones_v, acc_sh.at[didx_v], add=True)
        return carry

    lax.fori_loop(0, NCH, body, 0)
    plsc.subcore_barrier()
    pltpu.sync_copy(acc_sh.at[pl.ds(r0, ROWS_PT)],
                    out_hbm.at[c, pl.ds(r0, ROWS_PT)])


D_DEG = 16

_deg = pl.kernel(
    _deg_body,
    mesh=_MESH,
    compiler_params=_SC_PARAMS,
    out_type=jax.ShapeDtypeStruct((2, N_ACC, D_DEG), jnp.float32),
    scratch_types=[
        pltpu.VMEM((CH,), jnp.int32),
        pltpu.VMEM((CH, D_DEG), jnp.float32),
        pltpu.VMEM_SHARED((N_ACC, D_DEG), jnp.float32),
    ],
)


def _tc_b_body(x_ref, w1_ref, degp_ref, u1_ref, dinv_ref):
    deg = degp_ref[0, :N, 0:1] + degp_ref[1, :N, 0:1] + 1.0
    dinv = lax.rsqrt(deg)
    h1 = jnp.dot(x_ref[...], w1_ref[...], preferred_element_type=jnp.float32)
    u1_ref[:N, :] = h1 * dinv
    u1_ref[N:, :] = jnp.zeros((N_ACC - N, D_HID), jnp.float32)
    dinv_ref[...] = dinv


_tc_b = pl.pallas_call(
    _tc_b_body,
    out_shape=(
        jax.ShapeDtypeStruct((N_ACC, D_HID), jnp.float32),
        jax.ShapeDtypeStruct((N, 1), jnp.float32),
    ),
)


def _tc_d_body(p_ref, u1_ref, dinv_ref, b1_ref, u2_ref):
    t = p_ref[0, :N, :] + p_ref[1, :N, :] + u1_ref[:N, :]
    h = jnp.maximum(dinv_ref[...] * t + b1_ref[...], 0.0)
    u2_ref[:N, :] = dinv_ref[...] * h
    u2_ref[N:, :] = jnp.zeros((N_ACC - N, D_HID), jnp.float32)


_tc_d = pl.pallas_call(
    _tc_d_body,
    out_shape=jax.ShapeDtypeStruct((N_ACC, D_HID), jnp.float32),
)


def _tc_f_body(p_ref, u2_ref, dinv_ref, w2_ref, b2_ref, out_ref):
    g = dinv_ref[...] * (p_ref[0, :N, :] + p_ref[1, :N, :] + u2_ref[:N, :])
    out_ref[...] = jnp.dot(g, w2_ref[...],
                           preferred_element_type=jnp.float32) + b2_ref[...]


def kernel(x, edge_index, W1, b1, W2, b2):
    out_ch = W2.shape[1]
    tc_f = pl.pallas_call(
        _tc_f_body,
        out_shape=jax.ShapeDtypeStruct((N, out_ch), jnp.float32),
    )

    src = edge_index[0]
    dst = edge_index[1]
    pad = jnp.full((E_PAD - E,), N, dtype=jnp.int32)
    src3 = jnp.concatenate([src, pad]).reshape(NW, NCH, CH)
    dst3 = jnp.concatenate([dst, pad]).reshape(NW, NCH, CH)

    zeros32 = jnp.zeros((N_ACC, D_HID), jnp.float32)
    zeros_deg = jnp.zeros((N_ACC, D_DEG), jnp.float32)
    ones = jnp.ones((CH, D_DEG), jnp.float32)

    degp = _deg(dst3, ones, zeros_deg)
    u1, dinv = _tc_b(x, W1, degp)
    p1 = _prop(u1, src3, dst3, zeros32)
    u2 = _tc_d(p1, u1, dinv, b1.reshape(1, D_HID))
    p2 = _prop(u2, src3, dst3, zeros32)
    out = tc_f(p2, u2, dinv, W2, b2.reshape(1, out_ch))
    return out


# speedup vs baseline: 29.5215x; 1.7102x over previous
import functools

import jax
import jax.numpy as jnp
from jax import lax
from jax.experimental import pallas as pl
from jax.experimental.pallas import tpu as pltpu
from jax.experimental.pallas import tpu_sc as plsc

N = 10000
E = 320000
D_HID = 32

NW = 32
CH = 128
NCH = 80
E_PAD = NW * NCH * CH
ROWS_PT = 632
N_ACC = 16 * ROWS_PT

_MESH = plsc.VectorSubcoreMesh(core_axis_name="c", subcore_axis_name="s")


def _prop_body(u_hbm, src_hbm, dst_hbm, zeros_hbm, out_hbm,
               sidx_v, didx_v, rows0, rows1, acc_sh, gsem0, gsem1):
    c = lax.axis_index("c")
    s = lax.axis_index("s")
    wid = c * 16 + s
    r0 = s * ROWS_PT
    pltpu.sync_copy(zeros_hbm.at[pl.ds(r0, ROWS_PT)],
                    acc_sh.at[pl.ds(r0, ROWS_PT)])
    pltpu.sync_copy(src_hbm.at[wid], sidx_v)
    pltpu.sync_copy(dst_hbm.at[wid], didx_v)
    plsc.subcore_barrier()

    pltpu.async_copy(u_hbm.at[sidx_v.at[0]], rows0, gsem0)

    def group(g, carry):
        i0 = 2 * g
        i1 = i0 + 1
        pltpu.async_copy(u_hbm.at[sidx_v.at[i1]], rows1, gsem1)
        pltpu.make_async_copy(u_hbm.at[sidx_v.at[0]], rows0, gsem0).wait()
        pltpu.sync_copy(rows0, acc_sh.at[didx_v.at[i0]], add=True)
        nxt = jnp.minimum(i1 + 1, NCH - 1)
        pltpu.async_copy(u_hbm.at[sidx_v.at[nxt]], rows0, gsem0)
        pltpu.make_async_copy(u_hbm.at[sidx_v.at[0]], rows1, gsem1).wait()
        pltpu.sync_copy(rows1, acc_sh.at[didx_v.at[i1]], add=True)
        return carry

    lax.fori_loop(0, NCH // 2, group, 0)
    pltpu.make_async_copy(u_hbm.at[sidx_v.at[0]], rows0, gsem0).wait()
    plsc.subcore_barrier()
    pltpu.sync_copy(acc_sh.at[pl.ds(r0, ROWS_PT)],
                    out_hbm.at[c, pl.ds(r0, ROWS_PT)])


_SC_PARAMS = pltpu.CompilerParams(use_tc_tiling_on_sc=False)

_prop = pl.kernel(
    _prop_body,
    mesh=_MESH,
    compiler_params=_SC_PARAMS,
    out_type=jax.ShapeDtypeStruct((2, N_ACC, D_HID), jnp.float32),
    scratch_types=[
        pltpu.VMEM((NCH, CH), jnp.int32),
        pltpu.VMEM((NCH, CH), jnp.int32),
        pltpu.VMEM((CH, D_HID), jnp.float32),
        pltpu.VMEM((CH, D_HID), jnp.float32),
        pltpu.VMEM_SHARED((N_ACC, D_HID), jnp.float32),
        pltpu.SemaphoreType.DMA,
        pltpu.SemaphoreType.DMA,
    ],
)


def _deg_body(dst_hbm, ones_hbm, zeros_hbm, out_hbm, didx_v, ones_v, acc_sh,
              ssem):
    c = lax.axis_index("c")
    s = lax.axis_index("s")
    wid = c * 16 + s
    r0 = s * ROWS_PT
    pltpu.sync_copy(zeros_hbm.at[pl.ds(r0, ROWS_PT)],
                    acc_sh.at[pl.ds(r0, ROWS_PT)])
    pltpu.sync_copy(ones_hbm, ones_v)
    pltpu.sync_copy(dst_hbm.at[wid], didx_v)
    plsc.subcore_barrier()

    def fire(i, carry):
        pltpu.async_copy(ones_v, acc_sh.at[didx_v.at[i]], ssem, add=True)
        return carry

    lax.fori_loop(0, NCH, fire, 0)

    def drain(i, carry):
        pltpu.make_async_copy(ones_hbm, ones_v, ssem).wait()
        return carry

    lax.fori_loop(0, NCH, drain, 0)
    plsc.subcore_barrier()
    pltpu.sync_copy(acc_sh.at[pl.ds(r0, ROWS_PT)],
                    out_hbm.at[c, pl.ds(r0, ROWS_PT)])


D_DEG = 16

_deg = pl.kernel(
    _deg_body,
    mesh=_MESH,
    compiler_params=_SC_PARAMS,
    out_type=jax.ShapeDtypeStruct((2, N_ACC, D_DEG), jnp.float32),
    scratch_types=[
        pltpu.VMEM((NCH, CH), jnp.int32),
        pltpu.VMEM((CH, D_DEG), jnp.float32),
        pltpu.VMEM_SHARED((N_ACC, D_DEG), jnp.float32),
        pltpu.SemaphoreType.DMA,
    ],
)


def _tc_b_body(x_ref, w1_ref, degp_ref, u1_ref, dinv_ref):
    deg = degp_ref[0, :N, 0:1] + degp_ref[1, :N, 0:1] + 1.0
    dinv = lax.rsqrt(deg)
    h1 = jnp.dot(x_ref[...], w1_ref[...], preferred_element_type=jnp.float32)
    u1_ref[:N, :] = h1 * dinv
    u1_ref[N:, :] = jnp.zeros((N_ACC - N, D_HID), jnp.float32)
    dinv_ref[...] = dinv


_tc_b = pl.pallas_call(
    _tc_b_body,
    out_shape=(
        jax.ShapeDtypeStruct((N_ACC, D_HID), jnp.float32),
        jax.ShapeDtypeStruct((N, 1), jnp.float32),
    ),
)


def _tc_d_body(p_ref, u1_ref, dinv_ref, b1_ref, u2_ref):
    t = p_ref[0, :N, :] + p_ref[1, :N, :] + u1_ref[:N, :]
    h = jnp.maximum(dinv_ref[...] * t + b1_ref[...], 0.0)
    u2_ref[:N, :] = dinv_ref[...] * h
    u2_ref[N:, :] = jnp.zeros((N_ACC - N, D_HID), jnp.float32)


_tc_d = pl.pallas_call(
    _tc_d_body,
    out_shape=jax.ShapeDtypeStruct((N_ACC, D_HID), jnp.float32),
)


def _tc_f_body(p_ref, u2_ref, dinv_ref, w2_ref, b2_ref, out_ref):
    g = dinv_ref[...] * (p_ref[0, :N, :] + p_ref[1, :N, :] + u2_ref[:N, :])
    out_ref[...] = jnp.dot(g, w2_ref[...],
                           preferred_element_type=jnp.float32) + b2_ref[...]


def kernel(x, edge_index, W1, b1, W2, b2):
    out_ch = W2.shape[1]
    tc_f = pl.pallas_call(
        _tc_f_body,
        out_shape=jax.ShapeDtypeStruct((N, out_ch), jnp.float32),
    )

    src = edge_index[0]
    dst = edge_index[1]
    pad = jnp.full((E_PAD - E,), N, dtype=jnp.int32)
    src3 = jnp.concatenate([src, pad]).reshape(NW, NCH, CH)
    dst3 = jnp.concatenate([dst, pad]).reshape(NW, NCH, CH)

    zeros32 = jnp.zeros((N_ACC, D_HID), jnp.float32)
    zeros_deg = jnp.zeros((N_ACC, D_DEG), jnp.float32)
    ones = jnp.ones((CH, D_DEG), jnp.float32)

    degp = _deg(dst3, ones, zeros_deg)
    u1, dinv = _tc_b(x, W1, degp)
    p1 = _prop(u1, src3, dst3, zeros32)
    u2 = _tc_d(p1, u1, dinv, b1.reshape(1, D_HID))
    p2 = _prop(u2, src3, dst3, zeros32)
    out = tc_f(p2, u2, dinv, W2, b2.reshape(1, out_ch))
    return out


# speedup vs baseline: 30.6900x; 1.0396x over previous
import functools

import jax
import jax.numpy as jnp
from jax import lax
from jax.experimental import pallas as pl
from jax.experimental.pallas import tpu as pltpu
from jax.experimental.pallas import tpu_sc as plsc

N = 10000
E = 320000
D_HID = 32

NW = 32
CH = 128
NCH = 80
E_PAD = NW * NCH * CH
ROWS_PT = 632
N_ACC = 16 * ROWS_PT

_MESH = plsc.VectorSubcoreMesh(core_axis_name="c", subcore_axis_name="s")


def _prop_body(u_hbm, src_hbm, dst_hbm, zeros_hbm, out_hbm,
               sidx_v, didx_v, rows0, rows1, rows2, rows3, acc_sh,
               gsem0, gsem1, gsem2, gsem3, ssem0, ssem1, ssem2, ssem3):
    c = lax.axis_index("c")
    s = lax.axis_index("s")
    wid = c * 16 + s
    r0 = s * ROWS_PT
    pltpu.sync_copy(zeros_hbm.at[pl.ds(r0, ROWS_PT)],
                    acc_sh.at[pl.ds(r0, ROWS_PT)])
    pltpu.sync_copy(src_hbm.at[wid], sidx_v)
    pltpu.sync_copy(dst_hbm.at[wid], didx_v)
    plsc.subcore_barrier()

    rows = [rows0, rows1, rows2, rows3]
    gsem = [gsem0, gsem1, gsem2, gsem3]
    ssem = [ssem0, ssem1, ssem2, ssem3]

    def g_issue(i, b):
        pltpu.async_copy(u_hbm.at[sidx_v.at[i]], rows[b], gsem[b])

    def g_wait(b):
        pltpu.make_async_copy(u_hbm.at[sidx_v.at[0]], rows[b], gsem[b]).wait()

    def s_issue(i, b):
        pltpu.async_copy(rows[b], acc_sh.at[didx_v.at[i]], ssem[b], add=True)

    def s_wait(b):
        pltpu.make_async_copy(u_hbm.at[sidx_v.at[0]], rows[b], ssem[b]).wait()

    g_issue(0, 0)
    g_issue(1, 1)
    g_issue(2, 2)
    g_wait(0)
    s_issue(0, 0)
    g_issue(3, 3)
    g_wait(1)
    s_issue(1, 1)

    def group(g, carry):
        k0 = 4 * g + 2
        for b in range(4):
            k = k0 + b
            bb = (b + 2) % 4
            s_wait(b)
            g_issue(k + 2, b)
            g_wait(bb)
            s_issue(k, bb)
        return carry

    lax.fori_loop(0, (NCH - 4) // 4, group, 0)
    s_wait(0)
    g_wait(2)
    s_issue(NCH - 2, 2)
    s_wait(1)
    g_wait(3)
    s_issue(NCH - 1, 3)
    s_wait(2)
    s_wait(3)
    plsc.subcore_barrier()
    pltpu.sync_copy(acc_sh.at[pl.ds(r0, ROWS_PT)],
                    out_hbm.at[c, pl.ds(r0, ROWS_PT)])


_SC_PARAMS = pltpu.CompilerParams(use_tc_tiling_on_sc=False)

_prop = pl.kernel(
    _prop_body,
    mesh=_MESH,
    compiler_params=_SC_PARAMS,
    out_type=jax.ShapeDtypeStruct((2, N_ACC, D_HID), jnp.float32),
    scratch_types=[
        pltpu.VMEM((NCH, CH), jnp.int32),
        pltpu.VMEM((NCH, CH), jnp.int32),
        pltpu.VMEM((CH, D_HID), jnp.float32),
        pltpu.VMEM((CH, D_HID), jnp.float32),
        pltpu.VMEM((CH, D_HID), jnp.float32),
        pltpu.VMEM((CH, D_HID), jnp.float32),
        pltpu.VMEM_SHARED((N_ACC, D_HID), jnp.float32),
        pltpu.SemaphoreType.DMA,
        pltpu.SemaphoreType.DMA,
        pltpu.SemaphoreType.DMA,
        pltpu.SemaphoreType.DMA,
        pltpu.SemaphoreType.DMA,
        pltpu.SemaphoreType.DMA,
        pltpu.SemaphoreType.DMA,
        pltpu.SemaphoreType.DMA,
    ],
)


def _deg_body(dst_hbm, ones_hbm, zeros_hbm, out_hbm, didx_v, ones_v, acc_sh,
              ssem):
    c = lax.axis_index("c")
    s = lax.axis_index("s")
    wid = c * 16 + s
    r0 = s * ROWS_PT
    pltpu.sync_copy(zeros_hbm.at[pl.ds(r0, ROWS_PT)],
                    acc_sh.at[pl.ds(r0, ROWS_PT)])
    pltpu.sync_copy(ones_hbm, ones_v)
    pltpu.sync_copy(dst_hbm.at[wid], didx_v)
    plsc.subcore_barrier()

    def fire(i, carry):
        pltpu.async_copy(ones_v, acc_sh.at[didx_v.at[i]], ssem, add=True)
        return carry

    lax.fori_loop(0, NCH, fire, 0)

    def drain(i, carry):
        pltpu.make_async_copy(ones_hbm, ones_v, ssem).wait()
        return carry

    lax.fori_loop(0, NCH, drain, 0)
    plsc.subcore_barrier()
    pltpu.sync_copy(acc_sh.at[pl.ds(r0, ROWS_PT)],
                    out_hbm.at[c, pl.ds(r0, ROWS_PT)])


D_DEG = 16

_deg = pl.kernel(
    _deg_body,
    mesh=_MESH,
    compiler_params=_SC_PARAMS,
    out_type=jax.ShapeDtypeStruct((2, N_ACC, D_DEG), jnp.float32),
    scratch_types=[
        pltpu.VMEM((NCH, CH), jnp.int32),
        pltpu.VMEM((CH, D_DEG), jnp.float32),
        pltpu.VMEM_SHARED((N_ACC, D_DEG), jnp.float32),
        pltpu.SemaphoreType.DMA,
    ],
)


def _tc_b_body(x_ref, w1_ref, degp_ref, u1_ref, dinv_ref):
    deg = degp_ref[0, :N, 0:1] + degp_ref[1, :N, 0:1] + 1.0
    dinv = lax.rsqrt(deg)
    h1 = jnp.dot(x_ref[...], w1_ref[...], preferred_element_type=jnp.float32)
    u1_ref[:N, :] = h1 * dinv
    u1_ref[N:, :] = jnp.zeros((N_ACC - N, D_HID), jnp.float32)
    dinv_ref[...] = dinv


_tc_b = pl.pallas_call(
    _tc_b_body,
    out_shape=(
        jax.ShapeDtypeStruct((N_ACC, D_HID), jnp.float32),
        jax.ShapeDtypeStruct((N, 1), jnp.float32),
    ),
)


def _tc_d_body(p_ref, u1_ref, dinv_ref, b1_ref, u2_ref):
    t = p_ref[0, :N, :] + p_ref[1, :N, :] + u1_ref[:N, :]
    h = jnp.maximum(dinv_ref[...] * t + b1_ref[...], 0.0)
    u2_ref[:N, :] = dinv_ref[...] * h
    u2_ref[N:, :] = jnp.zeros((N_ACC - N, D_HID), jnp.float32)


_tc_d = pl.pallas_call(
    _tc_d_body,
    out_shape=jax.ShapeDtypeStruct((N_ACC, D_HID), jnp.float32),
)


def _tc_f_body(p_ref, u2_ref, dinv_ref, w2_ref, b2_ref, out_ref):
    g = dinv_ref[...] * (p_ref[0, :N, :] + p_ref[1, :N, :] + u2_ref[:N, :])
    out_ref[...] = jnp.dot(g, w2_ref[...],
                           preferred_element_type=jnp.float32) + b2_ref[...]


def kernel(x, edge_index, W1, b1, W2, b2):
    out_ch = W2.shape[1]
    tc_f = pl.pallas_call(
        _tc_f_body,
        out_shape=jax.ShapeDtypeStruct((N, out_ch), jnp.float32),
    )

    src = edge_index[0]
    dst = edge_index[1]
    pad = jnp.full((E_PAD - E,), N, dtype=jnp.int32)
    src3 = jnp.concatenate([src, pad]).reshape(NW, NCH, CH)
    dst3 = jnp.concatenate([dst, pad]).reshape(NW, NCH, CH)

    zeros32 = jnp.zeros((N_ACC, D_HID), jnp.float32)
    zeros_deg = jnp.zeros((N_ACC, D_DEG), jnp.float32)
    ones = jnp.ones((CH, D_DEG), jnp.float32)

    degp = _deg(dst3, ones, zeros_deg)
    u1, dinv = _tc_b(x, W1, degp)
    p1 = _prop(u1, src3, dst3, zeros32)
    u2 = _tc_d(p1, u1, dinv, b1.reshape(1, D_HID))
    p2 = _prop(u2, src3, dst3, zeros32)
    out = tc_f(p2, u2, dinv, W2, b2.reshape(1, out_ch))
    return out


# speedup vs baseline: 50.1786x; 1.6350x over previous
import functools

import jax
import jax.numpy as jnp
from jax import lax
from jax.experimental import pallas as pl
from jax.experimental.pallas import tpu as pltpu
from jax.experimental.pallas import tpu_sc as plsc

N = 10000
E = 320000
D_HID = 32

NW = 32
CH = 80
NCH = 125
ROWS_PT = 632
N_ACC = 16 * ROWS_PT

_MESH = plsc.VectorSubcoreMesh(core_axis_name="c", subcore_axis_name="s")


def _prop_body(u_hbm, src_hbm, dst_hbm, zeros_hbm, out_hbm,
               sidx_v, didx_v, rows0, rows1, rows2, rows3, acc_sh,
               gsem0, gsem1, gsem2, gsem3, ssem0, ssem1, ssem2, ssem3):
    c = lax.axis_index("c")
    s = lax.axis_index("s")
    wid = c * 16 + s
    r0 = s * ROWS_PT
    pltpu.sync_copy(zeros_hbm.at[pl.ds(r0, ROWS_PT)],
                    acc_sh.at[pl.ds(r0, ROWS_PT)])
    pltpu.sync_copy(src_hbm.at[wid], sidx_v)
    pltpu.sync_copy(dst_hbm.at[wid], didx_v)
    plsc.subcore_barrier()

    rows = [rows0, rows1, rows2, rows3]
    gsem = [gsem0, gsem1, gsem2, gsem3]
    ssem = [ssem0, ssem1, ssem2, ssem3]

    def g_issue(i, b):
        pltpu.async_copy(u_hbm.at[sidx_v.at[i]], rows[b], gsem[b])

    def g_wait(b):
        pltpu.make_async_copy(u_hbm.at[sidx_v.at[0]], rows[b], gsem[b]).wait()

    def s_issue(i, b):
        pltpu.async_copy(rows[b], acc_sh.at[didx_v.at[i]], ssem[b], add=True)

    def s_wait(b):
        pltpu.make_async_copy(u_hbm.at[sidx_v.at[0]], rows[b], ssem[b]).wait()

    g_issue(0, 0)
    g_issue(1, 1)
    g_issue(2, 2)
    g_wait(0)
    s_issue(0, 0)
    g_issue(3, 3)
    g_wait(1)
    s_issue(1, 1)

    def group(g, carry):
        k0 = 4 * g + 2
        for b in range(4):
            k = k0 + b
            bb = (b + 2) % 4
            s_wait(b)
            g_issue(k + 2, b)
            g_wait(bb)
            s_issue(k, bb)
        return carry

    lax.fori_loop(0, (NCH - 5) // 4, group, 0)
    s_wait(0)
    g_issue(NCH - 1, 0)
    g_wait(2)
    s_issue(NCH - 3, 2)
    s_wait(1)
    g_wait(3)
    s_issue(NCH - 2, 3)
    s_wait(2)
    g_wait(0)
    s_issue(NCH - 1, 0)
    s_wait(3)
    s_wait(0)
    plsc.subcore_barrier()
    pltpu.sync_copy(acc_sh.at[pl.ds(r0, ROWS_PT)],
                    out_hbm.at[c, pl.ds(r0, ROWS_PT)])


_SC_PARAMS = pltpu.CompilerParams(use_tc_tiling_on_sc=False)

_prop = pl.kernel(
    _prop_body,
    mesh=_MESH,
    compiler_params=_SC_PARAMS,
    out_type=jax.ShapeDtypeStruct((2, N_ACC, D_HID), jnp.float32),
    scratch_types=[
        pltpu.VMEM((NCH, CH), jnp.int32),
        pltpu.VMEM((NCH, CH), jnp.int32),
        pltpu.VMEM((CH, D_HID), jnp.float32),
        pltpu.VMEM((CH, D_HID), jnp.float32),
        pltpu.VMEM((CH, D_HID), jnp.float32),
        pltpu.VMEM((CH, D_HID), jnp.float32),
        pltpu.VMEM_SHARED((N_ACC, D_HID), jnp.float32),
        pltpu.SemaphoreType.DMA,
        pltpu.SemaphoreType.DMA,
        pltpu.SemaphoreType.DMA,
        pltpu.SemaphoreType.DMA,
        pltpu.SemaphoreType.DMA,
        pltpu.SemaphoreType.DMA,
        pltpu.SemaphoreType.DMA,
        pltpu.SemaphoreType.DMA,
    ],
)


def _deg_body(dst_hbm, ones_hbm, zeros_hbm, out_hbm, didx_v, ones_v, acc_sh,
              ssem):
    c = lax.axis_index("c")
    s = lax.axis_index("s")
    wid = c * 16 + s
    r0 = s * ROWS_PT
    pltpu.sync_copy(zeros_hbm.at[pl.ds(r0, ROWS_PT)],
                    acc_sh.at[pl.ds(r0, ROWS_PT)])
    pltpu.sync_copy(ones_hbm, ones_v)
    pltpu.sync_copy(dst_hbm.at[wid], didx_v)
    plsc.subcore_barrier()

    def fire(i, carry):
        pltpu.async_copy(ones_v, acc_sh.at[didx_v.at[i]], ssem, add=True)
        return carry

    lax.fori_loop(0, NCH, fire, 0)

    def drain(i, carry):
        pltpu.make_async_copy(ones_hbm, ones_v, ssem).wait()
        return carry

    lax.fori_loop(0, NCH, drain, 0)
    plsc.subcore_barrier()
    pltpu.sync_copy(acc_sh.at[pl.ds(r0, ROWS_PT)],
                    out_hbm.at[c, pl.ds(r0, ROWS_PT)])


D_DEG = 16

_deg = pl.kernel(
    _deg_body,
    mesh=_MESH,
    compiler_params=_SC_PARAMS,
    out_type=jax.ShapeDtypeStruct((2, N_ACC, D_DEG), jnp.float32),
    scratch_types=[
        pltpu.VMEM((NCH, CH), jnp.int32),
        pltpu.VMEM((CH, D_DEG), jnp.float32),
        pltpu.VMEM_SHARED((N_ACC, D_DEG), jnp.float32),
        pltpu.SemaphoreType.DMA,
    ],
)


def _tc_b0_body(x_ref, w1_ref, h1_ref):
    h1_ref[...] = jnp.dot(x_ref[...], w1_ref[...],
                          preferred_element_type=jnp.float32)


_tc_b0 = pl.pallas_call(
    _tc_b0_body,
    out_shape=jax.ShapeDtypeStruct((N, D_HID), jnp.float32),
)


def _tc_b1_body(h1_ref, degp_ref, u1_ref, dinv_ref):
    deg = degp_ref[0, :N, 0:1] + degp_ref[1, :N, 0:1] + 1.0
    dinv = lax.rsqrt(deg)
    u1_ref[:N, :] = h1_ref[...] * dinv
    u1_ref[N:, :] = jnp.zeros((N_ACC - N, D_HID), jnp.float32)
    dinv_ref[...] = dinv


_tc_b1 = pl.pallas_call(
    _tc_b1_body,
    out_shape=(
        jax.ShapeDtypeStruct((N_ACC, D_HID), jnp.float32),
        jax.ShapeDtypeStruct((N, 1), jnp.float32),
    ),
)


def _tc_d_body(p_ref, u1_ref, dinv_ref, b1_ref, u2_ref):
    t = p_ref[0, :N, :] + p_ref[1, :N, :] + u1_ref[:N, :]
    h = jnp.maximum(dinv_ref[...] * t + b1_ref[...], 0.0)
    u2_ref[:N, :] = dinv_ref[...] * h
    u2_ref[N:, :] = jnp.zeros((N_ACC - N, D_HID), jnp.float32)


_tc_d = pl.pallas_call(
    _tc_d_body,
    out_shape=jax.ShapeDtypeStruct((N_ACC, D_HID), jnp.float32),
)


def _tc_f_body(p_ref, u2_ref, dinv_ref, w2_ref, b2_ref, out_ref):
    g = dinv_ref[...] * (p_ref[0, :N, :] + p_ref[1, :N, :] + u2_ref[:N, :])
    out_ref[...] = jnp.dot(g, w2_ref[...],
                           preferred_element_type=jnp.float32) + b2_ref[...]


def kernel(x, edge_index, W1, b1, W2, b2):
    out_ch = W2.shape[1]
    tc_f = pl.pallas_call(
        _tc_f_body,
        out_shape=jax.ShapeDtypeStruct((N, out_ch), jnp.float32),
    )

    src3 = edge_index[0].reshape(NW, NCH, CH)
    dst3 = edge_index[1].reshape(NW, NCH, CH)

    zeros32 = jnp.zeros((N_ACC, D_HID), jnp.float32)
    zeros_deg = jnp.zeros((N_ACC, D_DEG), jnp.float32)
    ones = jnp.ones((CH, D_DEG), jnp.float32)

    degp = _deg(dst3, ones, zeros_deg)
    h1 = _tc_b0(x, W1)
    u1, dinv = _tc_b1(h1, degp)
    p1 = _prop(u1, src3, dst3, zeros32)
    u2 = _tc_d(p1, u1, dinv, b1.reshape(1, D_HID))
    p2 = _prop(u2, src3, dst3, zeros32)
    out = tc_f(p2, u2, dinv, W2, b2.reshape(1, out_ch))
    return out
